# baseline (speedup 1.0000x reference)
import jax
import jax.numpy as jnp
from jax.experimental import pallas as pl

B = 4096
KQ = 4096
D = 128
ADJ_K = 20
T = 0.1
EPS = 1e-07
N = B + KQ


def _l2norm_rows_pallas(x):
    def body(x_ref, o_ref):
        v = x_ref[...]
        n = jnp.sqrt(jnp.sum(v * v, axis=1, keepdims=True))
        o_ref[...] = v / n

    return pl.pallas_call(
        body,
        out_shape=jax.ShapeDtypeStruct(x.shape, x.dtype),
        grid=(x.shape[0] // 512,),
        in_specs=[pl.BlockSpec((512, x.shape[1]), lambda i: (i, 0))],
        out_specs=pl.BlockSpec((512, x.shape[1]), lambda i: (i, 0)),
    )(x)


def _cos_distance_softmax(x):
    soft = jax.nn.softmax(x, axis=2)
    w = jnp.linalg.norm(soft, ord=2, axis=2, keepdims=True)
    denom = jnp.maximum(w @ jnp.swapaxes(w, -1, -2), EPS)
    return 1.0 - (soft @ jnp.swapaxes(soft, -1, -2)) / denom


def _knn_edges(X, k):
    x = X[None, :, :]
    n = x.shape[1]
    dist = _cos_distance_softmax(x)[0]
    eye = jnp.eye(n, dtype=dist.dtype)
    dist = dist * (1.0 - eye) - eye
    _, idx = jax.lax.top_k(-dist, k)
    u = idx.reshape(-1)
    v = jnp.repeat(jnp.arange(n), k)
    return u, v


def _gcn_layer(h, u, v, W, b, n):
    deg_u = jnp.zeros((n,), dtype=h.dtype).at[u].add(1.0)
    deg_v = jnp.zeros((n,), dtype=h.dtype).at[v].add(1.0)
    norm_u = 1.0 / jnp.sqrt(jnp.maximum(deg_u, 1.0))
    norm_v = 1.0 / jnp.sqrt(jnp.maximum(deg_v, 1.0))
    m = h[u] * norm_u[u][:, None]
    agg = jax.ops.segment_sum(m, v, num_segments=n)
    return (agg * norm_v[:, None]) @ W + b


def _gnn(h, u, v, W1, b1, W2, b2, n):
    h = _gcn_layer(h, u, v, W1, b1, n)
    h = _gcn_layer(jax.nn.relu(h), u, v, W2, b2, n)
    return _l2norm_rows_pallas(h)


def _ce_loss0(logits):
    return jnp.mean(jax.nn.logsumexp(logits, axis=1) - logits[:, 0])


def kernel(im_q, im_k, queue, Wq1, bq1, Wq2, bq2, Wk1, bk1, Wk2, bk2):
    q = _l2norm_rows_pallas(im_q)
    k = _l2norm_rows_pallas(im_k)
    l_pos = jnp.sum(q * k, axis=1, keepdims=True)
    l_neg = q @ queue
    logits = jnp.concatenate([l_pos, l_neg], axis=1) / T
    loss_its = _ce_loss0(logits)
    Xq = jnp.concatenate([q, queue.T], axis=0)
    uq, vq = _knn_edges(Xq, ADJ_K)
    q_g = _gnn(Xq, uq, vq, Wq1, bq1, Wq2, bq2, N)[:B]
    q_g = q_g / jnp.linalg.norm(q_g, axis=1, keepdims=True)
    Xk = jnp.concatenate([k, queue.T], axis=0)
    uk, vk = _knn_edges(Xk, ADJ_K)
    k_g = _gnn(Xk, uk, vk, Wk1, bk1, Wk2, bk2, N)[:B]
    k_g = k_g / jnp.linalg.norm(k_g, axis=1, keepdims=True)
    l_pos_g = jnp.sum(q_g * k_g, axis=1, keepdims=True)
    l_neg_g = q_g @ queue
    logits_g = jnp.concatenate([l_pos_g, l_neg_g], axis=1) / T
    loss_gts = _ce_loss0(logits_g)
    loss = loss_its + loss_gts
    return (loss, loss_its, loss_gts)


# speedup vs baseline: 3.8374x; 3.8374x over previous
import functools

import jax
import jax.numpy as jnp
from jax import lax
from jax.experimental import pallas as pl
from jax.experimental.pallas import tpu as pltpu

ADJ_K = 20
T = 0.1
NEG_DIAG = -3.0e38
NEG_SEL = -2.0e38


def _row_normalize(x):
    n, d = x.shape
    blk = min(512, n)

    def body(x_ref, o_ref):
        v = x_ref[...]
        nrm = jnp.sqrt(jnp.sum(v * v, axis=1, keepdims=True))
        o_ref[...] = v / nrm

    return pl.pallas_call(
        body,
        out_shape=jax.ShapeDtypeStruct((n, d), x.dtype),
        grid=(n // blk,),
        in_specs=[pl.BlockSpec((blk, d), lambda i: (i, 0))],
        out_specs=pl.BlockSpec((blk, d), lambda i: (i, 0)),
    )(x)


def _softmax_rows(x):
    n, d = x.shape
    blk = min(512, n)

    def body(x_ref, o_ref, w_ref):
        v = x_ref[...]
        m = jnp.max(v, axis=1, keepdims=True)
        e = jnp.exp(v - m)
        s = e / jnp.sum(e, axis=1, keepdims=True)
        o_ref[...] = s
        w_ref[...] = jnp.sqrt(jnp.sum(s * s, axis=1, keepdims=True))

    return pl.pallas_call(
        body,
        out_shape=(
            jax.ShapeDtypeStruct((n, d), x.dtype),
            jax.ShapeDtypeStruct((n, 1), x.dtype),
        ),
        grid=(n // blk,),
        in_specs=[pl.BlockSpec((blk, d), lambda i: (i, 0))],
        out_specs=(
            pl.BlockSpec((blk, d), lambda i: (i, 0)),
            pl.BlockSpec((blk, 1), lambda i: (i, 0)),
        ),
    )(x)


def _knn_topk(soft, wc, wr, k):
    n, d = soft.shape
    blk = min(256, n)
    nblk = n // blk

    def body(xb_ref, xall_ref, wc_ref, wr_ref, idx_ref, deg_ref):
        i = pl.program_id(0)
        xb = xb_ref[...]
        sim = lax.dot_general(
            xb, xall_ref[...], (((1,), (1,)), ((), ())),
            preferred_element_type=jnp.float32)
        sim = sim / (wc_ref[...] * wr_ref[...])
        rows = jax.lax.broadcasted_iota(jnp.int32, (blk, 1), 0) + i * blk
        cols = jax.lax.broadcasted_iota(jnp.int32, (blk, n), 1)
        sim = jnp.where(cols == rows, NEG_DIAG, sim)
        picked = [rows]
        for _ in range(k - 1):
            m = jnp.max(sim, axis=1, keepdims=True)
            eq = sim >= m
            cand = jnp.where(eq, cols, n)
            j = jnp.min(cand, axis=1, keepdims=True)
            picked.append(j)
            sim = jnp.where(cols == j, NEG_SEL, sim)
        idx_ref[...] = jnp.concatenate(picked, axis=1)
        sel = jnp.sum(jnp.where(sim == NEG_SEL, 1.0, 0.0), axis=0,
                      keepdims=True)

        @pl.when(i == 0)
        def _():
            deg_ref[...] = jnp.zeros_like(deg_ref)

        deg_ref[...] += sel

    return pl.pallas_call(
        body,
        out_shape=(
            jax.ShapeDtypeStruct((n, k), jnp.int32),
            jax.ShapeDtypeStruct((1, n), jnp.float32),
        ),
        grid=(nblk,),
        in_specs=[
            pl.BlockSpec((blk, d), lambda i: (i, 0)),
            pl.BlockSpec((n, d), lambda i: (0, 0)),
            pl.BlockSpec((blk, 1), lambda i: (i, 0)),
            pl.BlockSpec((1, n), lambda i: (0, 0)),
        ],
        out_specs=(
            pl.BlockSpec((blk, k), lambda i: (i, 0)),
            pl.BlockSpec((1, n), lambda i: (0, 0)),
        ),
    )(soft, soft, wc, wr)


def _scale_rows(x, degn):
    n, d = x.shape
    blk = min(512, n)

    def body(x_ref, g_ref, o_ref):
        o_ref[...] = x_ref[...] * lax.rsqrt(g_ref[...] + 1.0)

    return pl.pallas_call(
        body,
        out_shape=jax.ShapeDtypeStruct((n, d), x.dtype),
        grid=(n // blk,),
        in_specs=[
            pl.BlockSpec((blk, d), lambda i: (i, 0)),
            pl.BlockSpec((blk, 1), lambda i: (i, 0)),
        ],
        out_specs=pl.BlockSpec((blk, d), lambda i: (i, 0)),
    )(x, degn)


def _gather_sum(table, idx):
    return jnp.sum(jnp.take(table, idx, axis=0), axis=1)


def _layer1(agg, w, b, degn):
    n, d = agg.shape
    blk = min(512, n)
    inv = float(ADJ_K) ** -0.5

    def body(a_ref, w_ref, b_ref, g_ref, o_ref):
        y = lax.dot_general(
            a_ref[...] * inv, w_ref[...], (((1,), (0,)), ((), ())),
            preferred_element_type=jnp.float32) + b_ref[...]
        o_ref[...] = jnp.maximum(y, 0.0) * lax.rsqrt(g_ref[...] + 1.0)

    return pl.pallas_call(
        body,
        out_shape=jax.ShapeDtypeStruct((n, d), jnp.float32),
        grid=(n // blk,),
        in_specs=[
            pl.BlockSpec((blk, d), lambda i: (i, 0)),
            pl.BlockSpec((d, d), lambda i: (0, 0)),
            pl.BlockSpec((1, d), lambda i: (0, 0)),
            pl.BlockSpec((blk, 1), lambda i: (i, 0)),
        ],
        out_specs=pl.BlockSpec((blk, d), lambda i: (i, 0)),
    )(agg, w, b.reshape(1, d), degn)


def _layer2(agg, w, b):
    n, d = agg.shape
    blk = min(512, n)
    inv = float(ADJ_K) ** -0.5

    def body(a_ref, w_ref, b_ref, o_ref):
        z = lax.dot_general(
            a_ref[...] * inv, w_ref[...], (((1,), (0,)), ((), ())),
            preferred_element_type=jnp.float32) + b_ref[...]
        nrm = jnp.sqrt(jnp.sum(z * z, axis=1, keepdims=True))
        o_ref[...] = z / nrm

    return pl.pallas_call(
        body,
        out_shape=jax.ShapeDtypeStruct((n, d), jnp.float32),
        grid=(n // blk,),
        in_specs=[
            pl.BlockSpec((blk, d), lambda i: (i, 0)),
            pl.BlockSpec((d, d), lambda i: (0, 0)),
            pl.BlockSpec((1, d), lambda i: (0, 0)),
        ],
        out_specs=pl.BlockSpec((blk, d), lambda i: (i, 0)),
    )(agg, w, b.reshape(1, d))


def _contrast_loss_sum(a, b, queue):
    bsz, d = a.shape
    kq = queue.shape[1]
    blk = min(512, bsz)
    nblk = bsz // blk

    def body(a_ref, b_ref, q_ref, o_ref):
        av = a_ref[...]
        lp = jnp.sum(av * b_ref[...], axis=1, keepdims=True) / T
        ln = lax.dot_general(
            av, q_ref[...], (((1,), (0,)), ((), ())),
            preferred_element_type=jnp.float32) / T
        m = jnp.maximum(jnp.max(ln, axis=1, keepdims=True), lp)
        s = jnp.sum(jnp.exp(ln - m), axis=1, keepdims=True) + jnp.exp(lp - m)
        lse = m + jnp.log(s)
        o_ref[...] = jnp.sum(lse - lp).reshape(1, 1, 1)

    part = pl.pallas_call(
        body,
        out_shape=jax.ShapeDtypeStruct((nblk, 1, 1), jnp.float32),
        grid=(nblk,),
        in_specs=[
            pl.BlockSpec((blk, d), lambda i: (i, 0)),
            pl.BlockSpec((blk, d), lambda i: (i, 0)),
            pl.BlockSpec((d, kq), lambda i: (0, 0)),
        ],
        out_specs=pl.BlockSpec((1, 1, 1), lambda i: (i, 0, 0)),
    )(a, b, queue)
    return jnp.sum(part)


def _graph_branch(x, soft, w, w1, b1, w2, b2):
    idx, deg = _knn_topk(soft, w, w.reshape(1, -1), ADJ_K)
    degn = deg.reshape(-1, 1)
    g1 = _scale_rows(x, degn)
    agg1 = _gather_sum(g1, idx)
    g2 = _layer1(agg1, w1, b1, degn)
    agg2 = _gather_sum(g2, idx)
    return _layer2(agg2, w2, b2)


def kernel(im_q, im_k, queue, Wq1, bq1, Wq2, bq2, Wk1, bk1, Wk2, bk2):
    bsz = im_q.shape[0]
    qt = queue.T
    qk = _row_normalize(jnp.concatenate([im_q, im_k], axis=0))
    q, k = qk[:bsz], qk[bsz:]
    soft, w = _softmax_rows(jnp.concatenate([q, k, qt], axis=0))
    sq, sk, sQ = soft[:bsz], soft[bsz:2 * bsz], soft[2 * bsz:]
    wq, wk, wQ = w[:bsz], w[bsz:2 * bsz], w[2 * bsz:]

    loss_its = _contrast_loss_sum(q, k, queue) / bsz

    xq = jnp.concatenate([q, qt], axis=0)
    xk = jnp.concatenate([k, qt], axis=0)
    fq = _graph_branch(xq, jnp.concatenate([sq, sQ], axis=0),
                       jnp.concatenate([wq, wQ], axis=0),
                       Wq1, bq1, Wq2, bq2)[:bsz]
    fk = _graph_branch(xk, jnp.concatenate([sk, sQ], axis=0),
                       jnp.concatenate([wk, wQ], axis=0),
                       Wk1, bk1, Wk2, bk2)[:bsz]
    loss_gts = _contrast_loss_sum(fq, fk, queue) / bsz

    loss = loss_its + loss_gts
    return (loss, loss_its, loss_gts)


# speedup vs baseline: 3.9632x; 1.0328x over previous
import functools

import jax
import jax.numpy as jnp
from jax import lax
from jax.experimental import pallas as pl
from jax.experimental.pallas import tpu as pltpu

ADJ_K = 20
T = 0.1
NEG_DIAG = -3.0e38
NEG_SEL = -2.0e38


def _row_normalize(x):
    n, d = x.shape
    blk = min(512, n)

    def body(x_ref, o_ref):
        v = x_ref[...]
        nrm = jnp.sqrt(jnp.sum(v * v, axis=1, keepdims=True))
        o_ref[...] = v / nrm

    return pl.pallas_call(
        body,
        out_shape=jax.ShapeDtypeStruct((n, d), x.dtype),
        grid=(n // blk,),
        in_specs=[pl.BlockSpec((blk, d), lambda i: (i, 0))],
        out_specs=pl.BlockSpec((blk, d), lambda i: (i, 0)),
    )(x)


def _softmax_rows(x):
    n, d = x.shape
    blk = min(512, n)

    def body(x_ref, o_ref, w_ref):
        v = x_ref[...]
        m = jnp.max(v, axis=1, keepdims=True)
        e = jnp.exp(v - m)
        s = e / jnp.sum(e, axis=1, keepdims=True)
        o_ref[...] = s
        w_ref[...] = jnp.sqrt(jnp.sum(s * s, axis=1, keepdims=True))

    return pl.pallas_call(
        body,
        out_shape=(
            jax.ShapeDtypeStruct((n, d), x.dtype),
            jax.ShapeDtypeStruct((n, 1), x.dtype),
        ),
        grid=(n // blk,),
        in_specs=[pl.BlockSpec((blk, d), lambda i: (i, 0))],
        out_specs=(
            pl.BlockSpec((blk, d), lambda i: (i, 0)),
            pl.BlockSpec((blk, 1), lambda i: (i, 0)),
        ),
    )(x)


def _knn_topk(soft, wc, wr, k):
    n, d = soft.shape
    blk = min(256, n)
    nblk = n // blk

    def body(xb_ref, xall_ref, wc_ref, wr_ref, idx_ref, deg_ref):
        i = pl.program_id(0)
        xb = xb_ref[...]
        sim = lax.dot_general(
            xb, xall_ref[...], (((1,), (1,)), ((), ())),
            preferred_element_type=jnp.float32)
        sim = sim / (wc_ref[...] * wr_ref[...])
        rows = jax.lax.broadcasted_iota(jnp.int32, (blk, 1), 0) + i * blk
        cols = jax.lax.broadcasted_iota(jnp.int32, (blk, n), 1)
        sim = jnp.where(cols == rows, NEG_DIAG, sim)
        picked = [rows]
        for _ in range(k - 1):
            j = jnp.argmax(sim, axis=1).astype(jnp.int32)[:, None]
            picked.append(j)
            sim = jnp.where(cols == j, NEG_SEL, sim)
        idx_ref[...] = jnp.concatenate(picked, axis=1)
        sel = jnp.sum(jnp.where(sim == NEG_SEL, 1.0, 0.0), axis=0,
                      keepdims=True)

        @pl.when(i == 0)
        def _():
            deg_ref[...] = jnp.zeros_like(deg_ref)

        deg_ref[...] += sel

    return pl.pallas_call(
        body,
        out_shape=(
            jax.ShapeDtypeStruct((n, k), jnp.int32),
            jax.ShapeDtypeStruct((1, n), jnp.float32),
        ),
        grid=(nblk,),
        in_specs=[
            pl.BlockSpec((blk, d), lambda i: (i, 0)),
            pl.BlockSpec((n, d), lambda i: (0, 0)),
            pl.BlockSpec((blk, 1), lambda i: (i, 0)),
            pl.BlockSpec((1, n), lambda i: (0, 0)),
        ],
        out_specs=(
            pl.BlockSpec((blk, k), lambda i: (i, 0)),
            pl.BlockSpec((1, n), lambda i: (0, 0)),
        ),
    )(soft, soft, wc, wr)


def _scale_rows(x, degn):
    n, d = x.shape
    blk = min(512, n)

    def body(x_ref, g_ref, o_ref):
        o_ref[...] = x_ref[...] * lax.rsqrt(g_ref[...] + 1.0)

    return pl.pallas_call(
        body,
        out_shape=jax.ShapeDtypeStruct((n, d), x.dtype),
        grid=(n // blk,),
        in_specs=[
            pl.BlockSpec((blk, d), lambda i: (i, 0)),
            pl.BlockSpec((blk, 1), lambda i: (i, 0)),
        ],
        out_specs=pl.BlockSpec((blk, d), lambda i: (i, 0)),
    )(x, degn)


def _gather_sum(table, idx):
    return jnp.sum(jnp.take(table, idx, axis=0), axis=1)


def _layer1(agg, w, b, degn):
    n, d = agg.shape
    blk = min(512, n)
    inv = float(ADJ_K) ** -0.5

    def body(a_ref, w_ref, b_ref, g_ref, o_ref):
        y = lax.dot_general(
            a_ref[...] * inv, w_ref[...], (((1,), (0,)), ((), ())),
            preferred_element_type=jnp.float32) + b_ref[...]
        o_ref[...] = jnp.maximum(y, 0.0) * lax.rsqrt(g_ref[...] + 1.0)

    return pl.pallas_call(
        body,
        out_shape=jax.ShapeDtypeStruct((n, d), jnp.float32),
        grid=(n // blk,),
        in_specs=[
            pl.BlockSpec((blk, d), lambda i: (i, 0)),
            pl.BlockSpec((d, d), lambda i: (0, 0)),
            pl.BlockSpec((1, d), lambda i: (0, 0)),
            pl.BlockSpec((blk, 1), lambda i: (i, 0)),
        ],
        out_specs=pl.BlockSpec((blk, d), lambda i: (i, 0)),
    )(agg, w, b.reshape(1, d), degn)


def _layer2(agg, w, b):
    n, d = agg.shape
    blk = min(512, n)
    inv = float(ADJ_K) ** -0.5

    def body(a_ref, w_ref, b_ref, o_ref):
        z = lax.dot_general(
            a_ref[...] * inv, w_ref[...], (((1,), (0,)), ((), ())),
            preferred_element_type=jnp.float32) + b_ref[...]
        nrm = jnp.sqrt(jnp.sum(z * z, axis=1, keepdims=True))
        o_ref[...] = z / nrm

    return pl.pallas_call(
        body,
        out_shape=jax.ShapeDtypeStruct((n, d), jnp.float32),
        grid=(n // blk,),
        in_specs=[
            pl.BlockSpec((blk, d), lambda i: (i, 0)),
            pl.BlockSpec((d, d), lambda i: (0, 0)),
            pl.BlockSpec((1, d), lambda i: (0, 0)),
        ],
        out_specs=pl.BlockSpec((blk, d), lambda i: (i, 0)),
    )(agg, w, b.reshape(1, d))


def _contrast_loss_sum(a, b, queue):
    bsz, d = a.shape
    kq = queue.shape[1]
    blk = min(512, bsz)
    nblk = bsz // blk

    def body(a_ref, b_ref, q_ref, o_ref):
        av = a_ref[...]
        lp = jnp.sum(av * b_ref[...], axis=1, keepdims=True) / T
        ln = lax.dot_general(
            av, q_ref[...], (((1,), (0,)), ((), ())),
            preferred_element_type=jnp.float32) / T
        m = jnp.maximum(jnp.max(ln, axis=1, keepdims=True), lp)
        s = jnp.sum(jnp.exp(ln - m), axis=1, keepdims=True) + jnp.exp(lp - m)
        lse = m + jnp.log(s)
        o_ref[...] = jnp.sum(lse - lp).reshape(1, 1, 1)

    part = pl.pallas_call(
        body,
        out_shape=jax.ShapeDtypeStruct((nblk, 1, 1), jnp.float32),
        grid=(nblk,),
        in_specs=[
            pl.BlockSpec((blk, d), lambda i: (i, 0)),
            pl.BlockSpec((blk, d), lambda i: (i, 0)),
            pl.BlockSpec((d, kq), lambda i: (0, 0)),
        ],
        out_specs=pl.BlockSpec((1, 1, 1), lambda i: (i, 0, 0)),
    )(a, b, queue)
    return jnp.sum(part)


def _graph_branch(x, soft, w, w1, b1, w2, b2):
    idx, deg = _knn_topk(soft, w, w.reshape(1, -1), ADJ_K)
    degn = deg.reshape(-1, 1)
    g1 = _scale_rows(x, degn)
    agg1 = _gather_sum(g1, idx)
    g2 = _layer1(agg1, w1, b1, degn)
    agg2 = _gather_sum(g2, idx)
    return _layer2(agg2, w2, b2)


def kernel(im_q, im_k, queue, Wq1, bq1, Wq2, bq2, Wk1, bk1, Wk2, bk2):
    bsz = im_q.shape[0]
    qt = queue.T
    qk = _row_normalize(jnp.concatenate([im_q, im_k], axis=0))
    q, k = qk[:bsz], qk[bsz:]
    soft, w = _softmax_rows(jnp.concatenate([q, k, qt], axis=0))
    sq, sk, sQ = soft[:bsz], soft[bsz:2 * bsz], soft[2 * bsz:]
    wq, wk, wQ = w[:bsz], w[bsz:2 * bsz], w[2 * bsz:]

    loss_its = _contrast_loss_sum(q, k, queue) / bsz

    xq = jnp.concatenate([q, qt], axis=0)
    xk = jnp.concatenate([k, qt], axis=0)
    fq = _graph_branch(xq, jnp.concatenate([sq, sQ], axis=0),
                       jnp.concatenate([wq, wQ], axis=0),
                       Wq1, bq1, Wq2, bq2)[:bsz]
    fk = _graph_branch(xk, jnp.concatenate([sk, sQ], axis=0),
                       jnp.concatenate([wk, wQ], axis=0),
                       Wk1, bk1, Wk2, bk2)[:bsz]
    loss_gts = _contrast_loss_sum(fq, fk, queue) / bsz

    loss = loss_its + loss_gts
    return (loss, loss_its, loss_gts)


# speedup vs baseline: 6.8470x; 1.7276x over previous
import functools

import jax
import jax.numpy as jnp
from jax import lax
from jax.experimental import pallas as pl
from jax.experimental.pallas import tpu as pltpu
from jax.experimental.pallas import tpu_sc as plsc

ADJ_K = 20
T = 0.1
NEG_DIAG = -3.0e38
NEG_SEL = -2.0e38


def _row_normalize(x):
    n, d = x.shape
    blk = min(512, n)

    def body(x_ref, o_ref):
        v = x_ref[...]
        nrm = jnp.sqrt(jnp.sum(v * v, axis=1, keepdims=True))
        o_ref[...] = v / nrm

    return pl.pallas_call(
        body,
        out_shape=jax.ShapeDtypeStruct((n, d), x.dtype),
        grid=(n // blk,),
        in_specs=[pl.BlockSpec((blk, d), lambda i: (i, 0))],
        out_specs=pl.BlockSpec((blk, d), lambda i: (i, 0)),
    )(x)


def _softmax_rows(x):
    n, d = x.shape
    blk = min(512, n)

    def body(x_ref, o_ref, w_ref):
        v = x_ref[...]
        m = jnp.max(v, axis=1, keepdims=True)
        e = jnp.exp(v - m)
        s = e / jnp.sum(e, axis=1, keepdims=True)
        o_ref[...] = s
        w_ref[...] = jnp.sqrt(jnp.sum(s * s, axis=1, keepdims=True))

    return pl.pallas_call(
        body,
        out_shape=(
            jax.ShapeDtypeStruct((n, d), x.dtype),
            jax.ShapeDtypeStruct((n, 1), x.dtype),
        ),
        grid=(n // blk,),
        in_specs=[pl.BlockSpec((blk, d), lambda i: (i, 0))],
        out_specs=(
            pl.BlockSpec((blk, d), lambda i: (i, 0)),
            pl.BlockSpec((blk, 1), lambda i: (i, 0)),
        ),
    )(x)


def _knn_topk(soft, wc, wr, k):
    n, d = soft.shape
    blk = min(256, n)
    nblk = n // blk

    def body(xb_ref, xall_ref, wc_ref, wr_ref, idx_ref, deg_ref):
        i = pl.program_id(0)
        xb = xb_ref[...]
        sim = lax.dot_general(
            xb, xall_ref[...], (((1,), (1,)), ((), ())),
            preferred_element_type=jnp.float32)
        sim = sim / (wc_ref[...] * wr_ref[...])
        rows = jax.lax.broadcasted_iota(jnp.int32, (blk, 1), 0) + i * blk
        cols = jax.lax.broadcasted_iota(jnp.int32, (blk, n), 1)
        sim = jnp.where(cols == rows, NEG_DIAG, sim)
        picked = [rows]
        for _ in range(k - 1):
            j = jnp.argmax(sim, axis=1).astype(jnp.int32)[:, None]
            picked.append(j)
            sim = jnp.where(cols == j, NEG_SEL, sim)
        idx_ref[...] = jnp.concatenate(picked, axis=1)
        sel = jnp.sum(jnp.where(sim == NEG_SEL, 1.0, 0.0), axis=0,
                      keepdims=True)

        @pl.when(i == 0)
        def _():
            deg_ref[...] = jnp.zeros_like(deg_ref)

        deg_ref[...] += sel

    return pl.pallas_call(
        body,
        out_shape=(
            jax.ShapeDtypeStruct((n, k), jnp.int32),
            jax.ShapeDtypeStruct((1, n), jnp.float32),
        ),
        grid=(nblk,),
        in_specs=[
            pl.BlockSpec((blk, d), lambda i: (i, 0)),
            pl.BlockSpec((n, d), lambda i: (0, 0)),
            pl.BlockSpec((blk, 1), lambda i: (i, 0)),
            pl.BlockSpec((1, n), lambda i: (0, 0)),
        ],
        out_specs=(
            pl.BlockSpec((blk, k), lambda i: (i, 0)),
            pl.BlockSpec((1, n), lambda i: (0, 0)),
        ),
    )(soft, soft, wc, wr)


def _scale_rows(x, degn):
    n, d = x.shape
    blk = min(512, n)

    def body(x_ref, g_ref, o_ref):
        o_ref[...] = x_ref[...] * lax.rsqrt(g_ref[...] + 1.0)

    return pl.pallas_call(
        body,
        out_shape=jax.ShapeDtypeStruct((n, d), x.dtype),
        grid=(n // blk,),
        in_specs=[
            pl.BlockSpec((blk, d), lambda i: (i, 0)),
            pl.BlockSpec((blk, 1), lambda i: (i, 0)),
        ],
        out_specs=pl.BlockSpec((blk, d), lambda i: (i, 0)),
    )(x, degn)


def _gather_sum(table, idx):
    n, d = table.shape
    k = idx.shape[1]
    info = plsc.get_sparse_core_info()
    nw = info.num_cores * info.num_subcores
    npw = n // nw
    ch = 32
    nch = npw // ch
    g = ch * k
    ng = g // 128
    idx3 = idx.reshape(n // ch, ng, 128)
    mesh = plsc.VectorSubcoreMesh(core_axis_name="c", subcore_axis_name="s")

    @functools.partial(
        pl.kernel, mesh=mesh,
        out_type=jax.ShapeDtypeStruct((n, d), jnp.float32),
        scratch_types=[
            pltpu.VMEM((ng, 128), jnp.int32),
            pltpu.VMEM((g, d), jnp.float32),
            pltpu.VMEM((ch, d), jnp.float32),
            pltpu.SemaphoreType.DMA,
        ],
    )
    def sc_body(table_hbm, idx_hbm, out_hbm, idx_v, rows_v, acc_v, sem):
        wid = lax.axis_index("s") * info.num_cores + lax.axis_index("c")
        base = wid * npw

        def node_body(u, carry):
            for lg in range(d // 16):
                sl = pl.ds(lg * 16, 16)
                acc = rows_v[u * k, sl]
                for s in range(1, k):
                    acc = acc + rows_v[u * k + s, sl]
                acc_v[u, sl] = acc
            return carry

        def chunk_body(ci, carry):
            gci = (base // ch) + ci
            pltpu.sync_copy(idx_hbm.at[gci], idx_v)
            copies = [
                pltpu.async_copy(table_hbm.at[idx_v.at[j]],
                                 rows_v.at[pl.ds(j * 128, 128)], sem)
                for j in range(ng)
            ]
            for cp in copies:
                cp.wait()
            lax.fori_loop(0, ch, node_body, 0)
            pltpu.sync_copy(acc_v, out_hbm.at[pl.ds(base + ci * ch, ch)])
            return carry

        lax.fori_loop(0, nch, chunk_body, 0)

    return sc_body(table, idx3)


def _layer1(agg, w, b, degn):
    n, d = agg.shape
    blk = min(512, n)
    inv = float(ADJ_K) ** -0.5

    def body(a_ref, w_ref, b_ref, g_ref, o_ref):
        y = lax.dot_general(
            a_ref[...] * inv, w_ref[...], (((1,), (0,)), ((), ())),
            preferred_element_type=jnp.float32) + b_ref[...]
        o_ref[...] = jnp.maximum(y, 0.0) * lax.rsqrt(g_ref[...] + 1.0)

    return pl.pallas_call(
        body,
        out_shape=jax.ShapeDtypeStruct((n, d), jnp.float32),
        grid=(n // blk,),
        in_specs=[
            pl.BlockSpec((blk, d), lambda i: (i, 0)),
            pl.BlockSpec((d, d), lambda i: (0, 0)),
            pl.BlockSpec((1, d), lambda i: (0, 0)),
            pl.BlockSpec((blk, 1), lambda i: (i, 0)),
        ],
        out_specs=pl.BlockSpec((blk, d), lambda i: (i, 0)),
    )(agg, w, b.reshape(1, d), degn)


def _layer2(agg, w, b):
    n, d = agg.shape
    blk = min(512, n)
    inv = float(ADJ_K) ** -0.5

    def body(a_ref, w_ref, b_ref, o_ref):
        z = lax.dot_general(
            a_ref[...] * inv, w_ref[...], (((1,), (0,)), ((), ())),
            preferred_element_type=jnp.float32) + b_ref[...]
        nrm = jnp.sqrt(jnp.sum(z * z, axis=1, keepdims=True))
        o_ref[...] = z / nrm

    return pl.pallas_call(
        body,
        out_shape=jax.ShapeDtypeStruct((n, d), jnp.float32),
        grid=(n // blk,),
        in_specs=[
            pl.BlockSpec((blk, d), lambda i: (i, 0)),
            pl.BlockSpec((d, d), lambda i: (0, 0)),
            pl.BlockSpec((1, d), lambda i: (0, 0)),
        ],
        out_specs=pl.BlockSpec((blk, d), lambda i: (i, 0)),
    )(agg, w, b.reshape(1, d))


def _contrast_loss_sum(a, b, queue):
    bsz, d = a.shape
    kq = queue.shape[1]
    blk = min(512, bsz)
    nblk = bsz // blk

    def body(a_ref, b_ref, q_ref, o_ref):
        av = a_ref[...]
        lp = jnp.sum(av * b_ref[...], axis=1, keepdims=True) / T
        ln = lax.dot_general(
            av, q_ref[...], (((1,), (0,)), ((), ())),
            preferred_element_type=jnp.float32) / T
        m = jnp.maximum(jnp.max(ln, axis=1, keepdims=True), lp)
        s = jnp.sum(jnp.exp(ln - m), axis=1, keepdims=True) + jnp.exp(lp - m)
        lse = m + jnp.log(s)
        o_ref[...] = jnp.sum(lse - lp).reshape(1, 1, 1)

    part = pl.pallas_call(
        body,
        out_shape=jax.ShapeDtypeStruct((nblk, 1, 1), jnp.float32),
        grid=(nblk,),
        in_specs=[
            pl.BlockSpec((blk, d), lambda i: (i, 0)),
            pl.BlockSpec((blk, d), lambda i: (i, 0)),
            pl.BlockSpec((d, kq), lambda i: (0, 0)),
        ],
        out_specs=pl.BlockSpec((1, 1, 1), lambda i: (i, 0, 0)),
    )(a, b, queue)
    return jnp.sum(part)


def _graph_branch(x, soft, w, w1, b1, w2, b2):
    idx, deg = _knn_topk(soft, w, w.reshape(1, -1), ADJ_K)
    degn = deg.reshape(-1, 1)
    g1 = _scale_rows(x, degn)
    agg1 = _gather_sum(g1, idx)
    g2 = _layer1(agg1, w1, b1, degn)
    agg2 = _gather_sum(g2, idx)
    return _layer2(agg2, w2, b2)


def kernel(im_q, im_k, queue, Wq1, bq1, Wq2, bq2, Wk1, bk1, Wk2, bk2):
    bsz = im_q.shape[0]
    qt = queue.T
    qk = _row_normalize(jnp.concatenate([im_q, im_k], axis=0))
    q, k = qk[:bsz], qk[bsz:]
    soft, w = _softmax_rows(jnp.concatenate([q, k, qt], axis=0))
    sq, sk, sQ = soft[:bsz], soft[bsz:2 * bsz], soft[2 * bsz:]
    wq, wk, wQ = w[:bsz], w[bsz:2 * bsz], w[2 * bsz:]

    loss_its = _contrast_loss_sum(q, k, queue) / bsz

    xq = jnp.concatenate([q, qt], axis=0)
    xk = jnp.concatenate([k, qt], axis=0)
    fq = _graph_branch(xq, jnp.concatenate([sq, sQ], axis=0),
                       jnp.concatenate([wq, wQ], axis=0),
                       Wq1, bq1, Wq2, bq2)[:bsz]
    fk = _graph_branch(xk, jnp.concatenate([sk, sQ], axis=0),
                       jnp.concatenate([wk, wQ], axis=0),
                       Wk1, bk1, Wk2, bk2)[:bsz]
    loss_gts = _contrast_loss_sum(fq, fk, queue) / bsz

    loss = loss_its + loss_gts
    return (loss, loss_its, loss_gts)


# speedup vs baseline: 7.6042x; 1.1106x over previous
import functools

import jax
import jax.numpy as jnp
from jax import lax
from jax.experimental import pallas as pl
from jax.experimental.pallas import tpu as pltpu
from jax.experimental.pallas import tpu_sc as plsc

ADJ_K = 20
T = 0.1
NEG_DIAG = -3.0e38
NEG_SEL = -2.0e38


def _row_normalize(x):
    n, d = x.shape
    blk = min(512, n)

    def body(x_ref, o_ref):
        v = x_ref[...]
        nrm = jnp.sqrt(jnp.sum(v * v, axis=1, keepdims=True))
        o_ref[...] = v / nrm

    return pl.pallas_call(
        body,
        out_shape=jax.ShapeDtypeStruct((n, d), x.dtype),
        grid=(n // blk,),
        in_specs=[pl.BlockSpec((blk, d), lambda i: (i, 0))],
        out_specs=pl.BlockSpec((blk, d), lambda i: (i, 0)),
    )(x)


def _softmax_rows(x):
    n, d = x.shape
    blk = min(512, n)

    def body(x_ref, o_ref, w_ref):
        v = x_ref[...]
        m = jnp.max(v, axis=1, keepdims=True)
        e = jnp.exp(v - m)
        s = e / jnp.sum(e, axis=1, keepdims=True)
        o_ref[...] = s
        w_ref[...] = jnp.sqrt(jnp.sum(s * s, axis=1, keepdims=True))

    return pl.pallas_call(
        body,
        out_shape=(
            jax.ShapeDtypeStruct((n, d), x.dtype),
            jax.ShapeDtypeStruct((n, 1), x.dtype),
        ),
        grid=(n // blk,),
        in_specs=[pl.BlockSpec((blk, d), lambda i: (i, 0))],
        out_specs=(
            pl.BlockSpec((blk, d), lambda i: (i, 0)),
            pl.BlockSpec((blk, 1), lambda i: (i, 0)),
        ),
    )(x)


def _knn_topk(soft, wc, wr, k):
    n, d = soft.shape
    blk = min(512, n)
    nblk = n // blk

    def body(xb_ref, xall_ref, wc_ref, wr_ref, idx_ref, deg_ref):
        i = pl.program_id(0)
        xb = xb_ref[...]
        sim = lax.dot_general(
            xb, xall_ref[...], (((1,), (1,)), ((), ())),
            preferred_element_type=jnp.float32)
        sim = sim / (wc_ref[...] * wr_ref[...])
        rows = jax.lax.broadcasted_iota(jnp.int32, (blk, 1), 0) + i * blk
        cols = jax.lax.broadcasted_iota(jnp.int32, (blk, n), 1)
        sim = jnp.where(cols == rows, NEG_DIAG, sim)
        picked = [rows]
        for _ in range(k - 1):
            j = jnp.argmax(sim, axis=1).astype(jnp.int32)[:, None]
            picked.append(j)
            sim = jnp.where(cols == j, NEG_SEL, sim)
        idx_ref[...] = jnp.concatenate(picked, axis=1)
        sel = jnp.sum(jnp.where(sim == NEG_SEL, 1.0, 0.0), axis=0,
                      keepdims=True)

        @pl.when(i == 0)
        def _():
            deg_ref[...] = jnp.zeros_like(deg_ref)

        deg_ref[...] += sel

    return pl.pallas_call(
        body,
        out_shape=(
            jax.ShapeDtypeStruct((n, k), jnp.int32),
            jax.ShapeDtypeStruct((1, n), jnp.float32),
        ),
        grid=(nblk,),
        in_specs=[
            pl.BlockSpec((blk, d), lambda i: (i, 0)),
            pl.BlockSpec((n, d), lambda i: (0, 0)),
            pl.BlockSpec((blk, 1), lambda i: (i, 0)),
            pl.BlockSpec((1, n), lambda i: (0, 0)),
        ],
        out_specs=(
            pl.BlockSpec((blk, k), lambda i: (i, 0)),
            pl.BlockSpec((1, n), lambda i: (0, 0)),
        ),
    )(soft, soft, wc, wr)


def _scale_rows(x, degn):
    n, d = x.shape
    blk = min(512, n)

    def body(x_ref, g_ref, o_ref):
        o_ref[...] = x_ref[...] * lax.rsqrt(g_ref[...] + 1.0)

    return pl.pallas_call(
        body,
        out_shape=jax.ShapeDtypeStruct((n, d), x.dtype),
        grid=(n // blk,),
        in_specs=[
            pl.BlockSpec((blk, d), lambda i: (i, 0)),
            pl.BlockSpec((blk, 1), lambda i: (i, 0)),
        ],
        out_specs=pl.BlockSpec((blk, d), lambda i: (i, 0)),
    )(x, degn)


def _gather_sum(table, idx):
    n, d = table.shape
    k = idx.shape[1]
    info = plsc.get_sparse_core_info()
    nw = info.num_cores * info.num_subcores
    npw = n // nw
    ch = 32
    nch = npw // ch
    g = ch * k
    ng = g // 128
    idx3 = idx.reshape(n // ch, ng, 128)
    mesh = plsc.VectorSubcoreMesh(core_axis_name="c", subcore_axis_name="s")

    @functools.partial(
        pl.kernel, mesh=mesh,
        out_type=jax.ShapeDtypeStruct((n, d), jnp.float32),
        scratch_types=[
            pltpu.VMEM((ng, 128), jnp.int32),
            pltpu.VMEM((g, d), jnp.float32),
            pltpu.VMEM((ch, d), jnp.float32),
            pltpu.SemaphoreType.DMA,
        ],
    )
    def sc_body(table_hbm, idx_hbm, out_hbm, idx_v, rows_v, acc_v, sem):
        wid = lax.axis_index("s") * info.num_cores + lax.axis_index("c")
        base = wid * npw

        def node_body(u, carry):
            for lg in range(d // 16):
                sl = pl.ds(lg * 16, 16)
                acc = rows_v[u * k, sl]
                for s in range(1, k):
                    acc = acc + rows_v[u * k + s, sl]
                acc_v[u, sl] = acc
            return carry

        def chunk_body(ci, carry):
            gci = (base // ch) + ci
            pltpu.sync_copy(idx_hbm.at[gci], idx_v)
            copies = [
                pltpu.async_copy(table_hbm.at[idx_v.at[j]],
                                 rows_v.at[pl.ds(j * 128, 128)], sem)
                for j in range(ng)
            ]
            for cp in copies:
                cp.wait()
            lax.fori_loop(0, ch, node_body, 0)
            pltpu.sync_copy(acc_v, out_hbm.at[pl.ds(base + ci * ch, ch)])
            return carry

        lax.fori_loop(0, nch, chunk_body, 0)

    return sc_body(table, idx3)


def _layer1(agg, w, b, degn):
    n, d = agg.shape
    blk = min(512, n)
    inv = float(ADJ_K) ** -0.5

    def body(a_ref, w_ref, b_ref, g_ref, o_ref):
        y = lax.dot_general(
            a_ref[...] * inv, w_ref[...], (((1,), (0,)), ((), ())),
            preferred_element_type=jnp.float32) + b_ref[...]
        o_ref[...] = jnp.maximum(y, 0.0) * lax.rsqrt(g_ref[...] + 1.0)

    return pl.pallas_call(
        body,
        out_shape=jax.ShapeDtypeStruct((n, d), jnp.float32),
        grid=(n // blk,),
        in_specs=[
            pl.BlockSpec((blk, d), lambda i: (i, 0)),
            pl.BlockSpec((d, d), lambda i: (0, 0)),
            pl.BlockSpec((1, d), lambda i: (0, 0)),
            pl.BlockSpec((blk, 1), lambda i: (i, 0)),
        ],
        out_specs=pl.BlockSpec((blk, d), lambda i: (i, 0)),
    )(agg, w, b.reshape(1, d), degn)


def _layer2(agg, w, b):
    n, d = agg.shape
    blk = min(512, n)
    inv = float(ADJ_K) ** -0.5

    def body(a_ref, w_ref, b_ref, o_ref):
        z = lax.dot_general(
            a_ref[...] * inv, w_ref[...], (((1,), (0,)), ((), ())),
            preferred_element_type=jnp.float32) + b_ref[...]
        nrm = jnp.sqrt(jnp.sum(z * z, axis=1, keepdims=True))
        o_ref[...] = z / nrm

    return pl.pallas_call(
        body,
        out_shape=jax.ShapeDtypeStruct((n, d), jnp.float32),
        grid=(n // blk,),
        in_specs=[
            pl.BlockSpec((blk, d), lambda i: (i, 0)),
            pl.BlockSpec((d, d), lambda i: (0, 0)),
            pl.BlockSpec((1, d), lambda i: (0, 0)),
        ],
        out_specs=pl.BlockSpec((blk, d), lambda i: (i, 0)),
    )(agg, w, b.reshape(1, d))


def _contrast_loss_sum(a, b, queue):
    bsz, d = a.shape
    kq = queue.shape[1]
    blk = min(512, bsz)
    nblk = bsz // blk

    def body(a_ref, b_ref, q_ref, o_ref):
        av = a_ref[...]
        lp = jnp.sum(av * b_ref[...], axis=1, keepdims=True) / T
        ln = lax.dot_general(
            av, q_ref[...], (((1,), (0,)), ((), ())),
            preferred_element_type=jnp.float32) / T
        m = jnp.maximum(jnp.max(ln, axis=1, keepdims=True), lp)
        s = jnp.sum(jnp.exp(ln - m), axis=1, keepdims=True) + jnp.exp(lp - m)
        lse = m + jnp.log(s)
        o_ref[...] = jnp.sum(lse - lp).reshape(1, 1, 1)

    part = pl.pallas_call(
        body,
        out_shape=jax.ShapeDtypeStruct((nblk, 1, 1), jnp.float32),
        grid=(nblk,),
        in_specs=[
            pl.BlockSpec((blk, d), lambda i: (i, 0)),
            pl.BlockSpec((blk, d), lambda i: (i, 0)),
            pl.BlockSpec((d, kq), lambda i: (0, 0)),
        ],
        out_specs=pl.BlockSpec((1, 1, 1), lambda i: (i, 0, 0)),
    )(a, b, queue)
    return jnp.sum(part)


def _two_graph_branches(xa, sa, wa, xb, sb, wb, wts_a, wts_b):
    idx_a, deg_a = _knn_topk(sa, wa, wa.reshape(1, -1), ADJ_K)
    idx_b, deg_b = _knn_topk(sb, wb, wb.reshape(1, -1), ADJ_K)
    dna, dnb = deg_a.reshape(-1, 1), deg_b.reshape(-1, 1)
    g1a = _scale_rows(xa, dna)
    g1b = _scale_rows(xb, dnb)
    agg1a = _gather_sum(g1a, idx_a)
    agg1b = _gather_sum(g1b, idx_b)
    w1a, b1a, w2a, b2a = wts_a
    w1b, b1b, w2b, b2b = wts_b
    g2a = _layer1(agg1a, w1a, b1a, dna)
    g2b = _layer1(agg1b, w1b, b1b, dnb)
    agg2a = _gather_sum(g2a, idx_a)
    agg2b = _gather_sum(g2b, idx_b)
    return _layer2(agg2a, w2a, b2a), _layer2(agg2b, w2b, b2b)


def kernel(im_q, im_k, queue, Wq1, bq1, Wq2, bq2, Wk1, bk1, Wk2, bk2):
    bsz = im_q.shape[0]
    qt = queue.T
    qk = _row_normalize(jnp.concatenate([im_q, im_k], axis=0))
    q, k = qk[:bsz], qk[bsz:]
    soft, w = _softmax_rows(jnp.concatenate([q, k, qt], axis=0))
    sq, sk, sQ = soft[:bsz], soft[bsz:2 * bsz], soft[2 * bsz:]
    wq, wk, wQ = w[:bsz], w[bsz:2 * bsz], w[2 * bsz:]

    loss_its = _contrast_loss_sum(q, k, queue) / bsz

    xq = jnp.concatenate([q, qt], axis=0)
    xk = jnp.concatenate([k, qt], axis=0)
    fq, fk = _two_graph_branches(
        xq, jnp.concatenate([sq, sQ], axis=0),
        jnp.concatenate([wq, wQ], axis=0),
        xk, jnp.concatenate([sk, sQ], axis=0),
        jnp.concatenate([wk, wQ], axis=0),
        (Wq1, bq1, Wq2, bq2), (Wk1, bk1, Wk2, bk2))
    fq, fk = fq[:bsz], fk[:bsz]
    loss_gts = _contrast_loss_sum(fq, fk, queue) / bsz

    loss = loss_its + loss_gts
    return (loss, loss_its, loss_gts)


# speedup vs baseline: 7.6056x; 1.0002x over previous
import functools

import jax
import jax.numpy as jnp
from jax import lax
from jax.experimental import pallas as pl
from jax.experimental.pallas import tpu as pltpu
from jax.experimental.pallas import tpu_sc as plsc

ADJ_K = 20
T = 0.1
NEG_DIAG = -3.0e38
NEG_SEL = -2.0e38


def _row_normalize(x):
    n, d = x.shape
    blk = min(512, n)

    def body(x_ref, o_ref):
        v = x_ref[...]
        nrm = jnp.sqrt(jnp.sum(v * v, axis=1, keepdims=True))
        o_ref[...] = v / nrm

    return pl.pallas_call(
        body,
        out_shape=jax.ShapeDtypeStruct((n, d), x.dtype),
        grid=(n // blk,),
        in_specs=[pl.BlockSpec((blk, d), lambda i: (i, 0))],
        out_specs=pl.BlockSpec((blk, d), lambda i: (i, 0)),
    )(x)


def _softmax_rows(x):
    n, d = x.shape
    blk = min(512, n)

    def body(x_ref, o_ref, w_ref):
        v = x_ref[...]
        m = jnp.max(v, axis=1, keepdims=True)
        e = jnp.exp(v - m)
        s = e / jnp.sum(e, axis=1, keepdims=True)
        o_ref[...] = s
        w_ref[...] = jnp.sqrt(jnp.sum(s * s, axis=1, keepdims=True))

    return pl.pallas_call(
        body,
        out_shape=(
            jax.ShapeDtypeStruct((n, d), x.dtype),
            jax.ShapeDtypeStruct((n, 1), x.dtype),
        ),
        grid=(n // blk,),
        in_specs=[pl.BlockSpec((blk, d), lambda i: (i, 0))],
        out_specs=(
            pl.BlockSpec((blk, d), lambda i: (i, 0)),
            pl.BlockSpec((blk, 1), lambda i: (i, 0)),
        ),
    )(x)


def _knn_topk(soft, wc, wr, k):
    n, d = soft.shape
    blk = min(512, n)
    nblk = n // blk

    def body(xb_ref, xall_ref, wc_ref, wr_ref, idx_ref, deg_ref):
        i = pl.program_id(0)
        xb = xb_ref[...]
        sim = lax.dot_general(
            xb, xall_ref[...], (((1,), (1,)), ((), ())),
            preferred_element_type=jnp.float32)
        sim = sim / (wc_ref[...] * wr_ref[...])
        rows = jax.lax.broadcasted_iota(jnp.int32, (blk, 1), 0) + i * blk
        cols = jax.lax.broadcasted_iota(jnp.int32, (blk, n), 1)
        sim = jnp.where(cols == rows, NEG_DIAG, sim)
        picked = [rows]
        for _ in range(k - 1):
            j = jnp.argmax(sim, axis=1).astype(jnp.int32)[:, None]
            picked.append(j)
            sim = jnp.where(cols == j, NEG_SEL, sim)
        idx_ref[...] = jnp.concatenate(picked, axis=1)
        sel = jnp.sum(jnp.where(sim == NEG_SEL, 1.0, 0.0), axis=0,
                      keepdims=True)

        @pl.when(i == 0)
        def _():
            deg_ref[...] = jnp.zeros_like(deg_ref)

        deg_ref[...] += sel

    return pl.pallas_call(
        body,
        out_shape=(
            jax.ShapeDtypeStruct((n, k), jnp.int32),
            jax.ShapeDtypeStruct((1, n), jnp.float32),
        ),
        grid=(nblk,),
        in_specs=[
            pl.BlockSpec((blk, d), lambda i: (i, 0)),
            pl.BlockSpec((n, d), lambda i: (0, 0)),
            pl.BlockSpec((blk, 1), lambda i: (i, 0)),
            pl.BlockSpec((1, n), lambda i: (0, 0)),
        ],
        out_specs=(
            pl.BlockSpec((blk, k), lambda i: (i, 0)),
            pl.BlockSpec((1, n), lambda i: (0, 0)),
        ),
    )(soft, soft, wc, wr)


def _scale_rows(x, degn):
    n, d = x.shape
    blk = min(512, n)

    def body(x_ref, g_ref, o_ref):
        o_ref[...] = x_ref[...] * lax.rsqrt(g_ref[...] + 1.0)

    return pl.pallas_call(
        body,
        out_shape=jax.ShapeDtypeStruct((n, d), x.dtype),
        grid=(n // blk,),
        in_specs=[
            pl.BlockSpec((blk, d), lambda i: (i, 0)),
            pl.BlockSpec((blk, 1), lambda i: (i, 0)),
        ],
        out_specs=pl.BlockSpec((blk, d), lambda i: (i, 0)),
    )(x, degn)


def _gather_sum(table, idx):
    n, d = table.shape
    k = idx.shape[1]
    info = plsc.get_sparse_core_info()
    nw = info.num_cores * info.num_subcores
    npw = n // nw
    ch = 32
    nch = npw // ch
    g = ch * k
    ng = g // 128
    idx3 = idx.reshape(n // ch, ng, 128)
    mesh = plsc.VectorSubcoreMesh(core_axis_name="c", subcore_axis_name="s")

    @functools.partial(
        pl.kernel, mesh=mesh,
        out_type=jax.ShapeDtypeStruct((n, d), jnp.float32),
        scratch_types=[
            pltpu.VMEM((ng, 128), jnp.int32),
            pltpu.VMEM((g, d), jnp.float32),
            pltpu.VMEM((ch, d), jnp.float32),
            pltpu.SemaphoreType.DMA,
        ],
    )
    def sc_body(table_hbm, idx_hbm, out_hbm, idx_v, rows_v, acc_v, sem):
        wid = lax.axis_index("s") * info.num_cores + lax.axis_index("c")
        base = wid * npw

        def node_body(u, carry):
            for lg in range(d // 16):
                sl = pl.ds(lg * 16, 16)
                acc = rows_v[u * k, sl]
                for s in range(1, k):
                    acc = acc + rows_v[u * k + s, sl]
                acc_v[u, sl] = acc
            return carry

        def chunk_body(ci, carry):
            gci = (base // ch) + ci
            pltpu.sync_copy(idx_hbm.at[gci], idx_v)
            copies = [
                pltpu.async_copy(table_hbm.at[idx_v.at[j]],
                                 rows_v.at[pl.ds(j * 128, 128)], sem)
                for j in range(ng)
            ]
            for cp in copies:
                cp.wait()
            lax.fori_loop(0, ch, node_body, 0)
            pltpu.sync_copy(acc_v, out_hbm.at[pl.ds(base + ci * ch, ch)])
            return carry

        lax.fori_loop(0, nch, chunk_body, 0)

    return sc_body(table, idx3)


def _layer1(agg, w, b, degn):
    n, d = agg.shape
    blk = min(512, n)
    inv = float(ADJ_K) ** -0.5

    def body(a_ref, w_ref, b_ref, g_ref, o_ref):
        y = lax.dot_general(
            a_ref[...] * inv, w_ref[...], (((1,), (0,)), ((), ())),
            preferred_element_type=jnp.float32) + b_ref[...]
        o_ref[...] = jnp.maximum(y, 0.0) * lax.rsqrt(g_ref[...] + 1.0)

    return pl.pallas_call(
        body,
        out_shape=jax.ShapeDtypeStruct((n, d), jnp.float32),
        grid=(n // blk,),
        in_specs=[
            pl.BlockSpec((blk, d), lambda i: (i, 0)),
            pl.BlockSpec((d, d), lambda i: (0, 0)),
            pl.BlockSpec((1, d), lambda i: (0, 0)),
            pl.BlockSpec((blk, 1), lambda i: (i, 0)),
        ],
        out_specs=pl.BlockSpec((blk, d), lambda i: (i, 0)),
    )(agg, w, b.reshape(1, d), degn)


def _layer2(agg, w, b):
    n, d = agg.shape
    blk = min(512, n)
    inv = float(ADJ_K) ** -0.5

    def body(a_ref, w_ref, b_ref, o_ref):
        z = lax.dot_general(
            a_ref[...] * inv, w_ref[...], (((1,), (0,)), ((), ())),
            preferred_element_type=jnp.float32) + b_ref[...]
        nrm = jnp.sqrt(jnp.sum(z * z, axis=1, keepdims=True))
        o_ref[...] = z / nrm

    return pl.pallas_call(
        body,
        out_shape=jax.ShapeDtypeStruct((n, d), jnp.float32),
        grid=(n // blk,),
        in_specs=[
            pl.BlockSpec((blk, d), lambda i: (i, 0)),
            pl.BlockSpec((d, d), lambda i: (0, 0)),
            pl.BlockSpec((1, d), lambda i: (0, 0)),
        ],
        out_specs=pl.BlockSpec((blk, d), lambda i: (i, 0)),
    )(agg, w, b.reshape(1, d))


def _contrast_loss_sum(a, b, queue):
    bsz, d = a.shape
    kq = queue.shape[1]
    blk = min(512, bsz)
    nblk = bsz // blk

    def body(a_ref, b_ref, q_ref, o_ref):
        av = a_ref[...]
        lp = jnp.sum(av * b_ref[...], axis=1, keepdims=True) / T
        ln = lax.dot_general(
            av, q_ref[...], (((1,), (0,)), ((), ())),
            preferred_element_type=jnp.float32) / T
        m = jnp.maximum(jnp.max(ln, axis=1, keepdims=True), lp)
        s = jnp.sum(jnp.exp(ln - m), axis=1, keepdims=True) + jnp.exp(lp - m)
        lse = m + jnp.log(s)
        o_ref[...] = jnp.sum(lse - lp).reshape(1, 1, 1)

    part = pl.pallas_call(
        body,
        out_shape=jax.ShapeDtypeStruct((nblk, 1, 1), jnp.float32),
        grid=(nblk,),
        in_specs=[
            pl.BlockSpec((blk, d), lambda i: (i, 0)),
            pl.BlockSpec((blk, d), lambda i: (i, 0)),
            pl.BlockSpec((d, kq), lambda i: (0, 0)),
        ],
        out_specs=pl.BlockSpec((1, 1, 1), lambda i: (i, 0, 0)),
    )(a, b, queue)
    return jnp.sum(part)


def _two_graph_branches(xa, sa, wa, xb, sb, wb, wts_a, wts_b, tc_filler):
    w1a, b1a, w2a, b2a = wts_a
    w1b, b1b, w2b, b2b = wts_b
    idx_a, deg_a = _knn_topk(sa, wa, wa.reshape(1, -1), ADJ_K)
    dna = deg_a.reshape(-1, 1)
    g1a = _scale_rows(xa, dna)
    agg1a = _gather_sum(g1a, idx_a)
    idx_b, deg_b = _knn_topk(sb, wb, wb.reshape(1, -1), ADJ_K)
    dnb = deg_b.reshape(-1, 1)
    g2a = _layer1(agg1a, w1a, b1a, dna)
    agg2a = _gather_sum(g2a, idx_a)
    g1b = _scale_rows(xb, dnb)
    filler = tc_filler()
    agg1b = _gather_sum(g1b, idx_b)
    fa = _layer2(agg2a, w2a, b2a)
    g2b = _layer1(agg1b, w1b, b1b, dnb)
    agg2b = _gather_sum(g2b, idx_b)
    return fa, _layer2(agg2b, w2b, b2b), filler


def kernel(im_q, im_k, queue, Wq1, bq1, Wq2, bq2, Wk1, bk1, Wk2, bk2):
    bsz = im_q.shape[0]
    qt = queue.T
    qk = _row_normalize(jnp.concatenate([im_q, im_k], axis=0))
    q, k = qk[:bsz], qk[bsz:]
    soft, w = _softmax_rows(jnp.concatenate([q, k, qt], axis=0))
    sq, sk, sQ = soft[:bsz], soft[bsz:2 * bsz], soft[2 * bsz:]
    wq, wk, wQ = w[:bsz], w[bsz:2 * bsz], w[2 * bsz:]

    xq = jnp.concatenate([q, qt], axis=0)
    xk = jnp.concatenate([k, qt], axis=0)
    fq, fk, its_sum = _two_graph_branches(
        xq, jnp.concatenate([sq, sQ], axis=0),
        jnp.concatenate([wq, wQ], axis=0),
        xk, jnp.concatenate([sk, sQ], axis=0),
        jnp.concatenate([wk, wQ], axis=0),
        (Wq1, bq1, Wq2, bq2), (Wk1, bk1, Wk2, bk2),
        lambda: _contrast_loss_sum(q, k, queue))
    loss_its = its_sum / bsz
    fq, fk = fq[:bsz], fk[:bsz]
    loss_gts = _contrast_loss_sum(fq, fk, queue) / bsz

    loss = loss_its + loss_gts
    return (loss, loss_its, loss_gts)
